# baseline (speedup 1.0000x reference)
import jax
import jax.numpy as jnp
from jax.experimental import pallas as pl
from jax.experimental.pallas import tpu as pltpu


def _gcn_body(x_ref, w_ref, adj_ref, sumrow_ref, y_ref, b_ref, out_ref,
              support_ref):
    @pl.when(pl.program_id(0) == 0)
    def _():
        support_ref[...] = jnp.dot(
            x_ref[...], w_ref[...], preferred_element_type=jnp.float32)

    agg = jnp.dot(
        adj_ref[...], support_ref[...], preferred_element_type=jnp.float32)
    out_ref[...] = jnp.maximum(
        agg / sumrow_ref[...] + y_ref[...] + b_ref[...], 0.0)


def kernel(x, y, adj, adj_sumrow, W, b):
    N, d_in = x.shape
    d_out = W.shape[1]
    BR = 1024
    b2 = b.reshape(1, d_out)
    return pl.pallas_call(
        _gcn_body,
        grid=(N // BR,),
        in_specs=[
            pl.BlockSpec((N, d_in), lambda i: (0, 0)),
            pl.BlockSpec((d_in, d_out), lambda i: (0, 0)),
            pl.BlockSpec((BR, N), lambda i: (i, 0)),
            pl.BlockSpec((BR, 1), lambda i: (i, 0)),
            pl.BlockSpec((BR, d_out), lambda i: (i, 0)),
            pl.BlockSpec((1, d_out), lambda i: (0, 0)),
        ],
        out_specs=pl.BlockSpec((BR, d_out), lambda i: (i, 0)),
        out_shape=jax.ShapeDtypeStruct((N, d_out), jnp.float32),
        scratch_shapes=[pltpu.VMEM((N, d_out), jnp.float32)],
    )(x, W, adj, adj_sumrow, y, b2)


# speedup vs baseline: 1.0851x; 1.0851x over previous
import jax
import jax.numpy as jnp
from jax.experimental import pallas as pl
from jax.experimental.pallas import tpu as pltpu


def _gcn_body(x_ref, w_ref, adj_ref, sumrow_ref, y_ref, b_ref, out_ref,
              support_ref):
    @pl.when(pl.program_id(0) == 0)
    def _():
        support_ref[...] = jnp.dot(
            x_ref[...], w_ref[...], preferred_element_type=jnp.float32)

    agg = jnp.dot(
        adj_ref[...], support_ref[...], preferred_element_type=jnp.float32)
    out_ref[...] = jnp.maximum(
        agg / sumrow_ref[...] + y_ref[...] + b_ref[...], 0.0)


def kernel(x, y, adj, adj_sumrow, W, b):
    N, d_in = x.shape
    d_out = W.shape[1]
    BR = 512
    b2 = b.reshape(1, d_out)
    return pl.pallas_call(
        _gcn_body,
        grid=(N // BR,),
        in_specs=[
            pl.BlockSpec((N, d_in), lambda i: (0, 0)),
            pl.BlockSpec((d_in, d_out), lambda i: (0, 0)),
            pl.BlockSpec((BR, N), lambda i: (i, 0)),
            pl.BlockSpec((BR, 1), lambda i: (i, 0)),
            pl.BlockSpec((BR, d_out), lambda i: (i, 0)),
            pl.BlockSpec((1, d_out), lambda i: (0, 0)),
        ],
        out_specs=pl.BlockSpec((BR, d_out), lambda i: (i, 0)),
        out_shape=jax.ShapeDtypeStruct((N, d_out), jnp.float32),
        scratch_shapes=[pltpu.VMEM((N, d_out), jnp.float32)],
    )(x, W, adj, adj_sumrow, y, b2)
